# baseline (speedup 1.0000x reference)
import functools
import math

import jax
import jax.numpy as jnp
from jax import lax
from jax.experimental import pallas as pl
from jax.experimental.pallas import tpu as pltpu
from jax.experimental.pallas import tpu_sc as plsc

_N = 50000
_E = 800000
_D = 64
_ED = 16
_G = 512

_EP = 819200
_EP2 = _EP // 2
_ECH = _EP // 128
_BE2 = 1024
_BN = 2000
_BN2 = _BN // 2
_ACC_R = 51200
_OPAD = 50048
_OP2 = _OPAD // 2
_ORW = 3128
_NP = 65536
_NCH = _NP // 128

_mesh = plsc.VectorSubcoreMesh(core_axis_name="c", subcore_axis_name="s")


def _leaky(x):
    return jnp.where(x >= 0, x, 0.15 * x)


def _sigmoid(x):
    return 1.0 / (1.0 + jnp.exp(-x))


def _bd(w):
    a, b = w.shape
    z = jnp.zeros((a, b), w.dtype)
    return jnp.concatenate(
        [jnp.concatenate([w, z], axis=1), jnp.concatenate([z, w], axis=1)],
        axis=0)


def _dup(b):
    return jnp.concatenate([b, b]).reshape(1, -1)


def _halfsum_mat(w):
    d = w
    return jnp.tile(jnp.eye(d, dtype=jnp.float32), (2, 2))


def _ln_pair(x, g, b, d):
    col = lax.broadcasted_iota(jnp.int32, x.shape, 1)
    s0 = jnp.sum(x[:, 0:d], axis=1, keepdims=True) * (1.0 / d)
    s1 = jnp.sum(x[:, d:2 * d], axis=1, keepdims=True) * (1.0 / d)
    m = jnp.where(col < d, s0, s1)
    dev = x - m
    v0 = jnp.sum(dev[:, 0:d] * dev[:, 0:d], axis=1, keepdims=True) * (1.0 / d)
    v1 = jnp.sum(dev[:, d:2 * d] * dev[:, d:2 * d], axis=1,
                 keepdims=True) * (1.0 / d)
    v = jnp.where(col < d, v0, v1)
    return dev / jnp.sqrt(v + 1e-5) * g + b


def _make_sc_gather():
    n_ch_tile = _ECH // 32
    n_sup = n_ch_tile // 8
    n_fire = 8

    @functools.partial(
        pl.kernel,
        mesh=_mesh,
        compiler_params=pltpu.CompilerParams(use_tc_tiling_on_sc=False),
        out_type=jax.ShapeDtypeStruct((_EP, _D), jnp.float32),
        scratch_types=[
            pltpu.VMEM((8, 128), jnp.int32),
            pltpu.VMEM((n_fire * 128, _D), jnp.float32),
            pltpu.SemaphoreType.DMA,
        ],
    )
    def k(table_hbm, idx_hbm, out_hbm, idx_v, rows_v, sem):
        c = lax.axis_index("c")
        s = lax.axis_index("s")
        ch_base = (c * 16 + s) * n_ch_tile

        def body(sup, carry):
            ch0 = ch_base + sup * 8
            pltpu.sync_copy(idx_hbm.at[pl.ds(ch0, 8)], idx_v)
            handles = [
                pltpu.async_copy(
                    table_hbm.at[idx_v.at[j]],
                    rows_v.at[pl.ds(j * 128, 128)],
                    sem,
                )
                for j in range(n_fire)
            ]
            for h in handles:
                h.wait()
            pltpu.sync_copy(
                rows_v, out_hbm.at[pl.ds(ch0 * 128, n_fire * 128)])
            return carry

        lax.fori_loop(0, n_sup, body, 0)

    return k


def _make_sc_scatter_pair():
    kb = 8
    n_ch_tile = _ECH // 32
    n_sup = n_ch_tile // kb
    zrows = _ACC_R // 16

    @functools.partial(
        pl.kernel,
        mesh=_mesh,
        compiler_params=pltpu.CompilerParams(use_tc_tiling_on_sc=False),
        out_type=jax.ShapeDtypeStruct((2, _OPAD, _D), jnp.float32),
        scratch_types=[
            pltpu.VMEM((kb, 128), jnp.int32),
            pltpu.VMEM((kb * 64, 16), jnp.float32),
            pltpu.VMEM((kb * 64, 16), jnp.float32),
            pltpu.VMEM_SHARED((_ACC_R, 16), jnp.float32),
            pltpu.SemaphoreType.DMA,
        ],
    )
    def k(vals_hbm, idx_hbm, zeros_hbm, out_hbm, idx_v, ve, vo, acc, sem):
        c = lax.axis_index("c")
        s = lax.axis_index("s")
        for q in range(4):
            pltpu.sync_copy(
                zeros_hbm.at[pl.ds(0, zrows)],
                acc.at[pl.ds(s * zrows, zrows)],
            )
            plsc.subcore_barrier()

            def body(sup, carry):
                ch0 = (c * 16 + s) * n_ch_tile + sup * kb
                pltpu.sync_copy(idx_hbm.at[pl.ds(ch0, kb)], idx_v)
                pr0 = ch0 * 64
                pltpu.sync_copy(
                    vals_hbm.at[pl.ds(pr0, kb * 64), pl.ds(q * 16, 16)], ve)
                pltpu.sync_copy(
                    vals_hbm.at[pl.ds(pr0, kb * 64), pl.ds(64 + q * 16, 16)],
                    vo)
                handles = []
                for j in range(kb):
                    handles.append(pltpu.async_copy(
                        ve.at[pl.ds(j * 64, 64)],
                        acc.at[idx_v.at[j, pl.ds(0, 64)]],
                        sem,
                        add=True,
                    ))
                    handles.append(pltpu.async_copy(
                        vo.at[pl.ds(j * 64, 64)],
                        acc.at[idx_v.at[j, pl.ds(64, 64)]],
                        sem,
                        add=True,
                    ))
                for h in handles:
                    h.wait()
                return carry

            lax.fori_loop(0, n_sup, body, 0)
            plsc.subcore_barrier()
            pltpu.sync_copy(
                acc.at[pl.ds(s * _ORW, _ORW)],
                out_hbm.at[c, pl.ds(s * _ORW, _ORW), pl.ds(q * 16, 16)],
            )
            if q < 3:
                plsc.subcore_barrier()

    return k


def _make_sc_pool():
    n_ch_tile = _NCH // 32
    gacc = _G + 16
    zrows = gacc // 16
    orows = _G // 16

    @functools.partial(
        pl.kernel,
        mesh=_mesh,
        compiler_params=pltpu.CompilerParams(use_tc_tiling_on_sc=False),
        out_type=[
            jax.ShapeDtypeStruct((2 * _G, 64), jnp.float32),
            jax.ShapeDtypeStruct((2 * _G, 16), jnp.float32),
        ],
        scratch_types=[
            pltpu.VMEM((8, 128), jnp.int32),
            pltpu.VMEM((8 * 128, 64), jnp.float32),
            pltpu.VMEM((128, 16), jnp.float32),
            pltpu.VMEM_SHARED((gacc, 64), jnp.float32),
            pltpu.VMEM_SHARED((gacc, 16), jnp.float32),
            pltpu.SemaphoreType.DMA,
        ],
    )
    def k(h_hbm, idx_hbm, zeros_hbm, ones_hbm, outh_hbm, outc_hbm,
          idx_v, vals_v, ones_v, acc_h, acc_c, sem):
        c = lax.axis_index("c")
        s = lax.axis_index("s")
        pltpu.sync_copy(ones_hbm, ones_v)
        pltpu.sync_copy(
            zeros_hbm.at[pl.ds(0, zrows), pl.ds(0, 64)],
            acc_h.at[pl.ds(s * zrows, zrows)],
        )
        pltpu.sync_copy(
            zeros_hbm.at[pl.ds(0, zrows), pl.ds(0, 16)],
            acc_c.at[pl.ds(s * zrows, zrows)],
        )
        plsc.subcore_barrier()

        for sup in range(n_ch_tile // 8):
            ch0 = (c * 16 + s) * n_ch_tile + sup * 8
            pltpu.sync_copy(idx_hbm.at[pl.ds(ch0, 8)], idx_v)
            pltpu.sync_copy(h_hbm.at[pl.ds(ch0 * 128, 8 * 128)], vals_v)
            handles = []
            for j in range(8):
                handles.append(pltpu.async_copy(
                    vals_v.at[pl.ds(j * 128, 128)],
                    acc_h.at[idx_v.at[j]],
                    sem,
                    add=True,
                ))
                handles.append(pltpu.async_copy(
                    ones_v,
                    acc_c.at[idx_v.at[j]],
                    sem,
                    add=True,
                ))
            for h in handles:
                h.wait()
        plsc.subcore_barrier()
        pltpu.sync_copy(
            acc_h.at[pl.ds(s * orows, orows)],
            outh_hbm.at[pl.ds(c * _G + s * orows, orows)],
        )
        pltpu.sync_copy(
            acc_c.at[pl.ds(s * orows, orows)],
            outc_hbm.at[pl.ds(c * _G + s * orows, orows)],
        )

    return k


_sc_gather64 = _make_sc_gather()
_sc_scatter = _make_sc_scatter_pair()
_sc_pool = _make_sc_pool()


def _full_spec(shape):
    nd = len(shape)
    return pl.BlockSpec(shape, lambda i=0, _n=nd: (0,) * _n)


def _tc_in(x2, w2c, b2c):
    def body(x_ref, w_ref, b_ref, o_ref):
        o_ref[...] = jnp.dot(
            x_ref[...], w_ref[...],
            preferred_element_type=jnp.float32, precision=lax.Precision.HIGHEST) + b_ref[...]

    return pl.pallas_call(
        body,
        grid=(_N // _BN,),
        in_specs=[
            pl.BlockSpec((_BN2, 2), lambda i: (i, 0)),
            _full_spec((2, 128)),
            _full_spec((1, 128)),
        ],
        out_specs=pl.BlockSpec((_BN2, 128), lambda i: (i, 0)),
        out_shape=jax.ShapeDtypeStruct((_N // 2, 128), jnp.float32),
    )(x2, w2c, b2c)


def _tc_gin_edge(ea, xg, wp):
    def body(ea_ref, xg_ref, w1, b1, lg, lb, w2, b2, w3, b3, o_ref):
        e = jnp.dot(ea_ref[...], w1[...], preferred_element_type=jnp.float32, precision=lax.Precision.HIGHEST)
        e = _ln_pair(e + b1[...], lg[...], lb[...], 64)
        e = _leaky(e)
        e = _leaky(jnp.dot(e, w2[...],
                           preferred_element_type=jnp.float32, precision=lax.Precision.HIGHEST) + b2[...])
        e = jnp.dot(e, w3[...], preferred_element_type=jnp.float32, precision=lax.Precision.HIGHEST) + b3[...]
        gate = _sigmoid(e)
        o_ref[...] = gate * xg_ref[...] + (1.0 - gate) * e

    return pl.pallas_call(
        body,
        grid=(_EP2 // _BE2,),
        in_specs=[
            pl.BlockSpec((_BE2, 128), lambda i: (i, 0)),
            pl.BlockSpec((_BE2, 128), lambda i: (i, 0)),
            _full_spec((128, 128)), _full_spec((1, 128)),
            _full_spec((1, 128)), _full_spec((1, 128)),
            _full_spec((128, 128)), _full_spec((1, 128)),
            _full_spec((128, 128)), _full_spec((1, 128)),
        ],
        out_specs=pl.BlockSpec((_BE2, 128), lambda i: (i, 0)),
        out_shape=jax.ShapeDtypeStruct((_EP2, 128), jnp.float32),
    )(ea, xg, *wp)


def _gin_edge_weights(p):
    enc = p['enc']
    w1 = jnp.zeros((128, 128), jnp.float32)
    w1 = w1.at[0:_ED, 0:64].set(enc['w1']).at[_ED:2 * _ED, 64:128].set(
        enc['w1'])
    return (w1, _dup(enc['b1']), _dup(enc['ln_g']), _dup(enc['ln_b']),
            _bd(enc['w2']), _dup(enc['b2']), _bd(enc['w3']), _dup(enc['b3']))


def _tc_gin_node(h, aggr, p):
    def body(h_ref, a_ref, eps, w1, b1, l1g, l1b, w2, b2, l2g, l2b, o_ref):
        x = h_ref[...]
        a = a_ref[0] + a_ref[1]
        t = (1.0 + eps[0, 0]) * x + a
        t = jnp.dot(t, w1[...], preferred_element_type=jnp.float32, precision=lax.Precision.HIGHEST) + b1[...]
        t = _leaky(_ln_pair(t, l1g[...], l1b[...], 128))
        t = jnp.dot(t, w2[...], preferred_element_type=jnp.float32, precision=lax.Precision.HIGHEST) + b2[...]
        t = _ln_pair(t, l2g[...], l2b[...], 64)
        o_ref[...] = x + t

    m = p['mlp']
    return pl.pallas_call(
        body,
        grid=(_N // _BN,),
        in_specs=[
            pl.BlockSpec((_BN2, 128), lambda i: (i, 0)),
            pl.BlockSpec((2, _BN2, 128), lambda i: (0, i, 0)),
            _full_spec((1, 1)),
            _full_spec((128, 256)), _full_spec((1, 256)),
            _full_spec((1, 256)), _full_spec((1, 256)),
            _full_spec((256, 128)), _full_spec((1, 128)),
            _full_spec((1, 128)), _full_spec((1, 128)),
        ],
        out_specs=pl.BlockSpec((_BN2, 128), lambda i: (i, 0)),
        out_shape=jax.ShapeDtypeStruct((_N // 2, 128), jnp.float32),
    )(h, aggr, p['eps'].reshape(1, 1),
      _bd(m['w1']), _dup(m['b1']), _dup(m['ln1_g']), _dup(m['ln1_b']),
      _bd(m['w2']), _dup(m['b2']), _dup(m['ln2_g']), _dup(m['ln2_b']))


def _tc_bn_leaky(h, g, b, res=None):
    nb = _N // _BN
    with_res = res is not None
    ksum = _halfsum_mat(64)

    def body(*refs):
        if with_res:
            h_ref, g_ref, b_ref, k_ref, r_ref, o_ref, acc = refs
        else:
            h_ref, g_ref, b_ref, k_ref, o_ref, acc = refs
        ph = pl.program_id(0)
        i = pl.program_id(1)

        @pl.when((ph == 0) & (i == 0))
        def _():
            acc[...] = jnp.zeros_like(acc)

        @pl.when(ph == 0)
        def _():
            x = h_ref[...]
            acc[0:1, :] += jnp.sum(x, axis=0, keepdims=True)
            acc[1:2, :] += jnp.sum(x * x, axis=0, keepdims=True)

        @pl.when(ph == 1)
        def _():
            x = h_ref[...]
            m = jnp.dot(acc[0:1, :], k_ref[...],
                        preferred_element_type=jnp.float32, precision=lax.Precision.HIGHEST) * (1.0 / _N)
            q = jnp.dot(acc[1:2, :], k_ref[...],
                        preferred_element_type=jnp.float32, precision=lax.Precision.HIGHEST) * (1.0 / _N)
            v = q - m * m
            y = _leaky((x - m) / jnp.sqrt(v + 1e-5) * g_ref[...] + b_ref[...])
            if with_res:
                y = y + r_ref[...]
            o_ref[...] = y

    in_specs = [
        pl.BlockSpec((_BN2, 128), lambda p, i: (i, 0)),
        pl.BlockSpec((1, 128), lambda p, i: (0, 0)),
        pl.BlockSpec((1, 128), lambda p, i: (0, 0)),
        pl.BlockSpec((128, 128), lambda p, i: (0, 0)),
    ]
    args = [h, _dup(g), _dup(b), ksum]
    if with_res:
        in_specs.append(pl.BlockSpec((_BN2, 128), lambda p, i: (i, 0)))
        args.append(res)
    return pl.pallas_call(
        body,
        grid=(2, nb),
        in_specs=in_specs,
        out_specs=pl.BlockSpec((_BN2, 128), lambda p, i: (i, 0)),
        out_shape=jax.ShapeDtypeStruct((_N // 2, 128), jnp.float32),
        scratch_shapes=[pltpu.VMEM((8, 128), jnp.float32)],
    )(*args)


def _tc_qkv(h, p):
    def body(h_ref, wq, bq, wk, bk, wv, bv, q_ref, k_ref, v_ref):
        x = h_ref[...]
        q_ref[...] = jnp.dot(
            x, wq[...], preferred_element_type=jnp.float32, precision=lax.Precision.HIGHEST) + bq[...]
        k_ref[...] = jnp.dot(
            x, wk[...], preferred_element_type=jnp.float32, precision=lax.Precision.HIGHEST) + bk[...]
        v_ref[...] = jnp.dot(
            x, wv[...], preferred_element_type=jnp.float32, precision=lax.Precision.HIGHEST) + bv[...]

    node_spec = pl.BlockSpec((_BN2, 128), lambda i: (i, 0))
    node_out = jax.ShapeDtypeStruct((_N // 2, 128), jnp.float32)
    return pl.pallas_call(
        body,
        grid=(_N // _BN,),
        in_specs=[
            node_spec,
            _full_spec((128, 128)), _full_spec((1, 128)),
            _full_spec((128, 128)), _full_spec((1, 128)),
            _full_spec((128, 128)), _full_spec((1, 128)),
        ],
        out_specs=[node_spec, node_spec, node_spec],
        out_shape=[node_out, node_out, node_out],
    )(h, _bd(p['wq']), _dup(p['bq']), _bd(p['wk']), _dup(p['bk']),
      _bd(p['wv']), _dup(p['bv']))


def _tc_tr_edge(ea, kg, vg, qg, p):
    isq = 1.0 / math.sqrt(_D // 2)
    gmat = (jnp.repeat(jnp.eye(4, dtype=jnp.float32), 32, axis=0) * isq)
    bmat = jnp.repeat(jnp.eye(4, dtype=jnp.float32), 32, axis=1)
    w1 = jnp.zeros((128, 128), jnp.float32)
    w1 = w1.at[0:_ED, 0:64].set(p['enc_w1']).at[_ED:2 * _ED, 64:128].set(
        p['enc_w1'])

    def body(ea_ref, kg_ref, vg_ref, qg_ref, w1r, b1r, w2r, b2r, wer, ber,
             g_ref, bb_ref, num_ref, den_ref):
        e = _leaky(jnp.dot(ea_ref[...], w1r[...],
                           preferred_element_type=jnp.float32, precision=lax.Precision.HIGHEST) + b1r[...])
        e = jnp.dot(e, w2r[...], preferred_element_type=jnp.float32, precision=lax.Precision.HIGHEST) + b2r[...]
        ek = jnp.dot(e, wer[...], preferred_element_type=jnp.float32, precision=lax.Precision.HIGHEST) + ber[...]
        kj = kg_ref[...] + ek
        qk = qg_ref[...] * kj
        logits = jnp.dot(qk, g_ref[...], preferred_element_type=jnp.float32, precision=lax.Precision.HIGHEST)
        u = jnp.exp(logits)
        ub = jnp.dot(u, bb_ref[...], preferred_element_type=jnp.float32, precision=lax.Precision.HIGHEST)
        num_ref[...] = (vg_ref[...] + ek) * ub
        den_ref[...] = ub

    edge_spec = pl.BlockSpec((_BE2, 128), lambda i: (i, 0))
    edge_out = jax.ShapeDtypeStruct((_EP2, 128), jnp.float32)
    return pl.pallas_call(
        body,
        grid=(_EP2 // _BE2,),
        in_specs=[
            edge_spec, edge_spec, edge_spec, edge_spec,
            _full_spec((128, 128)), _full_spec((1, 128)),
            _full_spec((128, 128)), _full_spec((1, 128)),
            _full_spec((128, 128)), _full_spec((1, 128)),
            _full_spec((128, 4)), _full_spec((4, 128)),
        ],
        out_specs=[edge_spec, edge_spec],
        out_shape=[edge_out, edge_out],
    )(ea, kg, vg, qg,
      w1, _dup(p['enc_b1']), _bd(p['enc_w2']), _dup(p['enc_b2']),
      _bd(p['we']), _dup(p['be']), gmat, bmat)


def _tc_tr_node(h, num, den, p):
    wb = p['wbeta']
    bb2 = jnp.repeat(jnp.eye(2, dtype=jnp.float32), 64, axis=1)

    def bd2(w):
        m = jnp.zeros((128, 2), jnp.float32)
        return m.at[0:64, 0:1].set(w).at[64:128, 1:2].set(w)

    def body(h_ref, n_ref, d_ref, wskip, bskip, wb0, wb1, wb2, bb_ref, o_ref):
        den_s = d_ref[0] + d_ref[1]
        out = (n_ref[0] + n_ref[1]) / (den_s + 1e-16)
        x_r = jnp.dot(h_ref[...], wskip[...],
                      preferred_element_type=jnp.float32, precision=lax.Precision.HIGHEST) + bskip[...]
        bl = (jnp.dot(out, wb0[...], preferred_element_type=jnp.float32, precision=lax.Precision.HIGHEST)
              + jnp.dot(x_r, wb1[...], preferred_element_type=jnp.float32, precision=lax.Precision.HIGHEST)
              + jnp.dot(out - x_r, wb2[...],
                        preferred_element_type=jnp.float32, precision=lax.Precision.HIGHEST))
        beta = _sigmoid(jnp.dot(bl, bb_ref[...],
                                preferred_element_type=jnp.float32, precision=lax.Precision.HIGHEST))
        o_ref[...] = beta * x_r + (1.0 - beta) * out

    return pl.pallas_call(
        body,
        grid=(_N // _BN,),
        in_specs=[
            pl.BlockSpec((_BN2, 128), lambda i: (i, 0)),
            pl.BlockSpec((2, _BN2, 128), lambda i: (0, i, 0)),
            pl.BlockSpec((2, _BN2, 128), lambda i: (0, i, 0)),
            _full_spec((128, 128)), _full_spec((1, 128)),
            _full_spec((128, 2)), _full_spec((128, 2)), _full_spec((128, 2)),
            _full_spec((2, 128)),
        ],
        out_specs=pl.BlockSpec((_BN2, 128), lambda i: (i, 0)),
        out_shape=jax.ShapeDtypeStruct((_N // 2, 128), jnp.float32),
    )(h, num, den, _bd(p['wskip']), _dup(p['bskip']),
      bd2(wb[0:_D]), bd2(wb[_D:2 * _D]), bd2(wb[2 * _D:3 * _D]), bb2)


def _tc_cls(sums, cnts, w1, b1, w2, b2):
    def body(s_ref, c_ref, w1r, b1r, w2r, b2r, o_ref):
        total = s_ref[pl.ds(0, _G)] + s_ref[pl.ds(_G, _G)]
        cnt = c_ref[pl.ds(0, _G), 0:1] + c_ref[pl.ds(_G, _G), 0:1]
        pooled = total / jnp.maximum(cnt, 1.0)
        z = _leaky(jnp.dot(pooled, w1r[...],
                           preferred_element_type=jnp.float32, precision=lax.Precision.HIGHEST) + b1r[...])
        o_ref[...] = jnp.dot(z, w2r[...],
                             preferred_element_type=jnp.float32, precision=lax.Precision.HIGHEST) + b2r[...]

    return pl.pallas_call(
        body,
        in_specs=[
            _full_spec((2 * _G, _D)), _full_spec((2 * _G, 16)),
            _full_spec((_D, _D // 2)), _full_spec((1, _D // 2)),
            _full_spec((_D // 2, 10)), _full_spec((1, 10)),
        ],
        out_specs=_full_spec((_G, 10)),
        out_shape=jax.ShapeDtypeStruct((_G, 10), jnp.float32),
    )(sums, cnts, w1, b1.reshape(1, -1), w2, b2.reshape(1, -1))


def _gin_layer(p, h, ea, src2, dst_eo, zeros16):
    xg = _sc_gather64(h.reshape(_N, _D), src2)
    msg = _tc_gin_edge(ea, xg.reshape(_EP2, 128), _gin_edge_weights(p))
    aggr = _sc_scatter(msg, dst_eo, zeros16)
    return _tc_gin_node(h, aggr.reshape(2, _OP2, 128), p)


def kernel(x, edge_index, edge_attr, batch, params):
    src = edge_index[0]
    dst = edge_index[1]
    pad_e = _EP - _E
    zpad = jnp.zeros((pad_e,), jnp.int32)
    src2 = jnp.concatenate([src, zpad]).reshape(_ECH, 128)
    dst2g = jnp.concatenate([dst, zpad]).reshape(_ECH, 128)
    dst_eo = (jnp.concatenate([dst, jnp.full((pad_e,), _N, jnp.int32)])
              .reshape(_ECH, 64, 2).transpose(0, 2, 1).reshape(_ECH, 128))
    ea = jnp.pad(edge_attr, ((0, pad_e), (0, 0))).reshape(_EP2, 2 * _ED)
    ea = jnp.pad(ea, ((0, 0), (0, 128 - 2 * _ED)))
    zeros16 = jnp.zeros((_ACC_R // 8, 128), jnp.float32).reshape(_ACC_R, 16)
    zeros64 = jnp.zeros((_ACC_R // 16, 64), jnp.float32)
    ones16 = jnp.ones((128, 16), jnp.float32)

    in_w = params['in_w']
    w2c = jnp.zeros((2, 128), jnp.float32)
    w2c = w2c.at[0, 0:64].set(in_w[0]).at[1, 64:128].set(in_w[0])
    h = _tc_in(x.reshape(_N // 2, 2), w2c, _dup(params['in_b']))

    h = _gin_layer(params['gin1_0'], h, ea, src2, dst_eo, zeros16)
    h = _gin_layer(params['gin1_1'], h, ea, src2, dst_eo, zeros16)
    h = _tc_bn_leaky(h, params['bn1_g'], params['bn1_b'])
    res = h

    tr = params['tr']
    q, k, v = _tc_qkv(h, tr)
    qg = _sc_gather64(q.reshape(_N, _D), dst2g)
    kg = _sc_gather64(k.reshape(_N, _D), src2)
    vg = _sc_gather64(v.reshape(_N, _D), src2)
    num_e, den_e = _tc_tr_edge(ea, kg.reshape(_EP2, 128),
                               vg.reshape(_EP2, 128),
                               qg.reshape(_EP2, 128), tr)
    num = _sc_scatter(num_e, dst_eo, zeros16)
    den = _sc_scatter(den_e, dst_eo, zeros16)
    h = _tc_tr_node(h, num.reshape(2, _OP2, 128),
                    den.reshape(2, _OP2, 128), tr)
    h = _tc_bn_leaky(h, params['bntr_g'], params['bntr_b'], res)

    h = _gin_layer(params['gin2_0'], h, ea, src2, dst_eo, zeros16)
    h = _gin_layer(params['gin2_1'], h, ea, src2, dst_eo, zeros16)
    h = _tc_bn_leaky(h, params['bn2_g'], params['bn2_b'])

    hp = jnp.pad(h.reshape(_N, _D), ((0, _NP - _N), (0, 0)))
    bp = jnp.concatenate(
        [batch, jnp.full((_NP - _N,), _G, jnp.int32)]).reshape(_NCH, 128)
    sums, cnts = _sc_pool(hp, bp, zeros64, ones16)
    return _tc_cls(sums, cnts,
                   params['cls_w1'], params['cls_b1'],
                   params['cls_w2'], params['cls_b2'])


# speedup vs baseline: 1.2952x; 1.2952x over previous
import functools
import math

import jax
import jax.numpy as jnp
from jax import lax
from jax.experimental import pallas as pl
from jax.experimental.pallas import tpu as pltpu
from jax.experimental.pallas import tpu_sc as plsc

_N = 50000
_E = 800000
_D = 64
_ED = 16
_G = 512

_EP = 819200
_EP2 = _EP // 2
_ECH = _EP // 128
_BE2 = 1024
_BN = 2000
_BN2 = _BN // 2
_ACC_R = 51200
_OPAD = 50048
_OP2 = _OPAD // 2
_ORW = 3128
_NP = 65536
_NCH = _NP // 128

_mesh = plsc.VectorSubcoreMesh(core_axis_name="c", subcore_axis_name="s")


def _leaky(x):
    return jnp.where(x >= 0, x, 0.15 * x)


def _sigmoid(x):
    return 1.0 / (1.0 + jnp.exp(-x))


def _bd(w):
    a, b = w.shape
    z = jnp.zeros((a, b), w.dtype)
    return jnp.concatenate(
        [jnp.concatenate([w, z], axis=1), jnp.concatenate([z, w], axis=1)],
        axis=0)


def _dup(b):
    return jnp.concatenate([b, b]).reshape(1, -1)


def _halfsum_mat(w):
    d = w
    return jnp.tile(jnp.eye(d, dtype=jnp.float32), (2, 2))


def _ln_pair(x, g, b, d):
    col = lax.broadcasted_iota(jnp.int32, x.shape, 1)
    s0 = jnp.sum(x[:, 0:d], axis=1, keepdims=True) * (1.0 / d)
    s1 = jnp.sum(x[:, d:2 * d], axis=1, keepdims=True) * (1.0 / d)
    m = jnp.where(col < d, s0, s1)
    dev = x - m
    v0 = jnp.sum(dev[:, 0:d] * dev[:, 0:d], axis=1, keepdims=True) * (1.0 / d)
    v1 = jnp.sum(dev[:, d:2 * d] * dev[:, d:2 * d], axis=1,
                 keepdims=True) * (1.0 / d)
    v = jnp.where(col < d, v0, v1)
    return dev / jnp.sqrt(v + 1e-5) * g + b


def _make_sc_gather():
    n_ch_tile = _ECH // 32
    n_sup = n_ch_tile // 8
    n_fire = 8

    @functools.partial(
        pl.kernel,
        mesh=_mesh,
        compiler_params=pltpu.CompilerParams(use_tc_tiling_on_sc=False),
        out_type=jax.ShapeDtypeStruct((_EP, _D), jnp.float32),
        scratch_types=[
            pltpu.VMEM((8, 128), jnp.int32),
            pltpu.VMEM((n_fire * 128, _D), jnp.float32),
            pltpu.SemaphoreType.DMA,
        ],
    )
    def k(table_hbm, idx_hbm, out_hbm, idx_v, rows_v, sem):
        c = lax.axis_index("c")
        s = lax.axis_index("s")
        ch_base = (c * 16 + s) * n_ch_tile

        def body(sup, carry):
            ch0 = ch_base + sup * 8
            pltpu.sync_copy(idx_hbm.at[pl.ds(ch0, 8)], idx_v)
            handles = [
                pltpu.async_copy(
                    table_hbm.at[idx_v.at[j]],
                    rows_v.at[pl.ds(j * 128, 128)],
                    sem,
                )
                for j in range(n_fire)
            ]
            for h in handles:
                h.wait()
            pltpu.sync_copy(
                rows_v, out_hbm.at[pl.ds(ch0 * 128, n_fire * 128)])
            return carry

        lax.fori_loop(0, n_sup, body, 0)

    return k


def _make_sc_scatter_pair():
    kb = 8
    n_ch_tile = _ECH // 32
    n_sup = n_ch_tile // kb
    zrows = _ACC_R // 16

    @functools.partial(
        pl.kernel,
        mesh=_mesh,
        compiler_params=pltpu.CompilerParams(use_tc_tiling_on_sc=False),
        out_type=jax.ShapeDtypeStruct((2, _OPAD, _D), jnp.float32),
        scratch_types=[
            pltpu.VMEM((kb, 128), jnp.int32),
            pltpu.VMEM((kb * 64, 16), jnp.float32),
            pltpu.VMEM((kb * 64, 16), jnp.float32),
            pltpu.VMEM_SHARED((_ACC_R, 16), jnp.float32),
            pltpu.SemaphoreType.DMA,
        ],
    )
    def k(vals_hbm, idx_hbm, zeros_hbm, out_hbm, idx_v, ve, vo, acc, sem):
        c = lax.axis_index("c")
        s = lax.axis_index("s")
        for q in range(4):
            pltpu.sync_copy(
                zeros_hbm.at[pl.ds(0, zrows)],
                acc.at[pl.ds(s * zrows, zrows)],
            )
            plsc.subcore_barrier()

            def body(sup, carry):
                ch0 = (c * 16 + s) * n_ch_tile + sup * kb
                pltpu.sync_copy(idx_hbm.at[pl.ds(ch0, kb)], idx_v)
                pr0 = ch0 * 64
                pltpu.sync_copy(
                    vals_hbm.at[pl.ds(pr0, kb * 64), pl.ds(q * 16, 16)], ve)
                pltpu.sync_copy(
                    vals_hbm.at[pl.ds(pr0, kb * 64), pl.ds(64 + q * 16, 16)],
                    vo)
                handles = []
                for j in range(kb):
                    handles.append(pltpu.async_copy(
                        ve.at[pl.ds(j * 64, 64)],
                        acc.at[idx_v.at[j, pl.ds(0, 64)]],
                        sem,
                        add=True,
                    ))
                    handles.append(pltpu.async_copy(
                        vo.at[pl.ds(j * 64, 64)],
                        acc.at[idx_v.at[j, pl.ds(64, 64)]],
                        sem,
                        add=True,
                    ))
                for h in handles:
                    h.wait()
                return carry

            lax.fori_loop(0, n_sup, body, 0)
            plsc.subcore_barrier()
            pltpu.sync_copy(
                acc.at[pl.ds(s * _ORW, _ORW)],
                out_hbm.at[c, pl.ds(s * _ORW, _ORW), pl.ds(q * 16, 16)],
            )
            if q < 3:
                plsc.subcore_barrier()

    return k


def _make_sc_pool():
    n_ch_tile = _NCH // 32
    gacc = _G + 16
    zrows = gacc // 16
    orows = _G // 16

    @functools.partial(
        pl.kernel,
        mesh=_mesh,
        compiler_params=pltpu.CompilerParams(use_tc_tiling_on_sc=False),
        out_type=[
            jax.ShapeDtypeStruct((2 * _G, 64), jnp.float32),
            jax.ShapeDtypeStruct((2 * _G, 16), jnp.float32),
        ],
        scratch_types=[
            pltpu.VMEM((8, 128), jnp.int32),
            pltpu.VMEM((8 * 128, 64), jnp.float32),
            pltpu.VMEM((128, 16), jnp.float32),
            pltpu.VMEM_SHARED((gacc, 64), jnp.float32),
            pltpu.VMEM_SHARED((gacc, 16), jnp.float32),
            pltpu.SemaphoreType.DMA,
        ],
    )
    def k(h_hbm, idx_hbm, zeros_hbm, ones_hbm, outh_hbm, outc_hbm,
          idx_v, vals_v, ones_v, acc_h, acc_c, sem):
        c = lax.axis_index("c")
        s = lax.axis_index("s")
        pltpu.sync_copy(ones_hbm, ones_v)
        pltpu.sync_copy(
            zeros_hbm.at[pl.ds(0, zrows), pl.ds(0, 64)],
            acc_h.at[pl.ds(s * zrows, zrows)],
        )
        pltpu.sync_copy(
            zeros_hbm.at[pl.ds(0, zrows), pl.ds(0, 16)],
            acc_c.at[pl.ds(s * zrows, zrows)],
        )
        plsc.subcore_barrier()

        for sup in range(n_ch_tile // 8):
            ch0 = (c * 16 + s) * n_ch_tile + sup * 8
            pltpu.sync_copy(idx_hbm.at[pl.ds(ch0, 8)], idx_v)
            pltpu.sync_copy(h_hbm.at[pl.ds(ch0 * 128, 8 * 128)], vals_v)
            handles = []
            for j in range(8):
                handles.append(pltpu.async_copy(
                    vals_v.at[pl.ds(j * 128, 128)],
                    acc_h.at[idx_v.at[j]],
                    sem,
                    add=True,
                ))
                handles.append(pltpu.async_copy(
                    ones_v,
                    acc_c.at[idx_v.at[j]],
                    sem,
                    add=True,
                ))
            for h in handles:
                h.wait()
        plsc.subcore_barrier()
        pltpu.sync_copy(
            acc_h.at[pl.ds(s * orows, orows)],
            outh_hbm.at[pl.ds(c * _G + s * orows, orows)],
        )
        pltpu.sync_copy(
            acc_c.at[pl.ds(s * orows, orows)],
            outc_hbm.at[pl.ds(c * _G + s * orows, orows)],
        )

    return k


_sc_gather64 = _make_sc_gather()
_sc_scatter = _make_sc_scatter_pair()
_sc_pool = _make_sc_pool()


def _full_spec(shape):
    nd = len(shape)
    return pl.BlockSpec(shape, lambda i=0, _n=nd: (0,) * _n)


def _tc_in(x2, w2c, b2c):
    def body(x_ref, w_ref, b_ref, o_ref):
        o_ref[...] = jnp.dot(
            x_ref[...], w_ref[...],
            preferred_element_type=jnp.float32, precision=lax.Precision.HIGHEST) + b_ref[...]

    return pl.pallas_call(
        body,
        grid=(_N // _BN,),
        in_specs=[
            pl.BlockSpec((_BN2, 2), lambda i: (i, 0)),
            _full_spec((2, 128)),
            _full_spec((1, 128)),
        ],
        out_specs=pl.BlockSpec((_BN2, 128), lambda i: (i, 0)),
        out_shape=jax.ShapeDtypeStruct((_N // 2, 128), jnp.float32),
    )(x2, w2c, b2c)


def _tc_gin_edge(ea, xg, wp):
    def body(ea_ref, xg_ref, w1, b1, lg, lb, w2, b2, w3, b3, o_ref):
        e = jnp.dot(ea_ref[...], w1[...], preferred_element_type=jnp.float32)
        e = _ln_pair(e + b1[...], lg[...], lb[...], 64)
        e = _leaky(e)
        e = _leaky(jnp.dot(e, w2[...],
                           preferred_element_type=jnp.float32) + b2[...])
        e = jnp.dot(e, w3[...], preferred_element_type=jnp.float32) + b3[...]
        gate = _sigmoid(e)
        o_ref[...] = gate * xg_ref[...] + (1.0 - gate) * e

    return pl.pallas_call(
        body,
        grid=(_EP2 // _BE2,),
        in_specs=[
            pl.BlockSpec((_BE2, 128), lambda i: (i, 0)),
            pl.BlockSpec((_BE2, 128), lambda i: (i, 0)),
            _full_spec((128, 128)), _full_spec((1, 128)),
            _full_spec((1, 128)), _full_spec((1, 128)),
            _full_spec((128, 128)), _full_spec((1, 128)),
            _full_spec((128, 128)), _full_spec((1, 128)),
        ],
        out_specs=pl.BlockSpec((_BE2, 128), lambda i: (i, 0)),
        out_shape=jax.ShapeDtypeStruct((_EP2, 128), jnp.float32),
    )(ea, xg, *wp)


def _gin_edge_weights(p):
    enc = p['enc']
    w1 = jnp.zeros((128, 128), jnp.float32)
    w1 = w1.at[0:_ED, 0:64].set(enc['w1']).at[_ED:2 * _ED, 64:128].set(
        enc['w1'])
    return (w1, _dup(enc['b1']), _dup(enc['ln_g']), _dup(enc['ln_b']),
            _bd(enc['w2']), _dup(enc['b2']), _bd(enc['w3']), _dup(enc['b3']))


def _tc_gin_node(h, aggr, p):
    def body(h_ref, a_ref, eps, w1, b1, l1g, l1b, w2, b2, l2g, l2b, o_ref):
        x = h_ref[...]
        a = a_ref[0] + a_ref[1]
        t = (1.0 + eps[0, 0]) * x + a
        t = jnp.dot(t, w1[...], preferred_element_type=jnp.float32) + b1[...]
        t = _leaky(_ln_pair(t, l1g[...], l1b[...], 128))
        t = jnp.dot(t, w2[...], preferred_element_type=jnp.float32) + b2[...]
        t = _ln_pair(t, l2g[...], l2b[...], 64)
        o_ref[...] = x + t

    m = p['mlp']
    return pl.pallas_call(
        body,
        grid=(_N // _BN,),
        in_specs=[
            pl.BlockSpec((_BN2, 128), lambda i: (i, 0)),
            pl.BlockSpec((2, _BN2, 128), lambda i: (0, i, 0)),
            _full_spec((1, 1)),
            _full_spec((128, 256)), _full_spec((1, 256)),
            _full_spec((1, 256)), _full_spec((1, 256)),
            _full_spec((256, 128)), _full_spec((1, 128)),
            _full_spec((1, 128)), _full_spec((1, 128)),
        ],
        out_specs=pl.BlockSpec((_BN2, 128), lambda i: (i, 0)),
        out_shape=jax.ShapeDtypeStruct((_N // 2, 128), jnp.float32),
    )(h, aggr, p['eps'].reshape(1, 1),
      _bd(m['w1']), _dup(m['b1']), _dup(m['ln1_g']), _dup(m['ln1_b']),
      _bd(m['w2']), _dup(m['b2']), _dup(m['ln2_g']), _dup(m['ln2_b']))


def _tc_bn_leaky(h, g, b, res=None):
    nb = _N // _BN
    with_res = res is not None
    ksum = _halfsum_mat(64)

    def body(*refs):
        if with_res:
            h_ref, g_ref, b_ref, k_ref, r_ref, o_ref, acc = refs
        else:
            h_ref, g_ref, b_ref, k_ref, o_ref, acc = refs
        ph = pl.program_id(0)
        i = pl.program_id(1)

        @pl.when((ph == 0) & (i == 0))
        def _():
            acc[...] = jnp.zeros_like(acc)

        @pl.when(ph == 0)
        def _():
            x = h_ref[...]
            acc[0:1, :] += jnp.sum(x, axis=0, keepdims=True)
            acc[1:2, :] += jnp.sum(x * x, axis=0, keepdims=True)

        @pl.when(ph == 1)
        def _():
            x = h_ref[...]
            m = jnp.dot(acc[0:1, :], k_ref[...],
                        preferred_element_type=jnp.float32, precision=lax.Precision.HIGHEST) * (1.0 / _N)
            q = jnp.dot(acc[1:2, :], k_ref[...],
                        preferred_element_type=jnp.float32, precision=lax.Precision.HIGHEST) * (1.0 / _N)
            v = q - m * m
            y = _leaky((x - m) / jnp.sqrt(v + 1e-5) * g_ref[...] + b_ref[...])
            if with_res:
                y = y + r_ref[...]
            o_ref[...] = y

    in_specs = [
        pl.BlockSpec((_BN2, 128), lambda p, i: (i, 0)),
        pl.BlockSpec((1, 128), lambda p, i: (0, 0)),
        pl.BlockSpec((1, 128), lambda p, i: (0, 0)),
        pl.BlockSpec((128, 128), lambda p, i: (0, 0)),
    ]
    args = [h, _dup(g), _dup(b), ksum]
    if with_res:
        in_specs.append(pl.BlockSpec((_BN2, 128), lambda p, i: (i, 0)))
        args.append(res)
    return pl.pallas_call(
        body,
        grid=(2, nb),
        in_specs=in_specs,
        out_specs=pl.BlockSpec((_BN2, 128), lambda p, i: (i, 0)),
        out_shape=jax.ShapeDtypeStruct((_N // 2, 128), jnp.float32),
        scratch_shapes=[pltpu.VMEM((8, 128), jnp.float32)],
    )(*args)


def _tc_qkv(h, p):
    def body(h_ref, wq, bq, wk, bk, wv, bv, q_ref, k_ref, v_ref):
        x = h_ref[...]
        q_ref[...] = jnp.dot(
            x, wq[...], preferred_element_type=jnp.float32) + bq[...]
        k_ref[...] = jnp.dot(
            x, wk[...], preferred_element_type=jnp.float32) + bk[...]
        v_ref[...] = jnp.dot(
            x, wv[...], preferred_element_type=jnp.float32) + bv[...]

    node_spec = pl.BlockSpec((_BN2, 128), lambda i: (i, 0))
    node_out = jax.ShapeDtypeStruct((_N // 2, 128), jnp.float32)
    return pl.pallas_call(
        body,
        grid=(_N // _BN,),
        in_specs=[
            node_spec,
            _full_spec((128, 128)), _full_spec((1, 128)),
            _full_spec((128, 128)), _full_spec((1, 128)),
            _full_spec((128, 128)), _full_spec((1, 128)),
        ],
        out_specs=[node_spec, node_spec, node_spec],
        out_shape=[node_out, node_out, node_out],
    )(h, _bd(p['wq']), _dup(p['bq']), _bd(p['wk']), _dup(p['bk']),
      _bd(p['wv']), _dup(p['bv']))


def _tc_tr_edge(ea, kg, vg, qg, p):
    isq = 1.0 / math.sqrt(_D // 2)
    gmat = (jnp.repeat(jnp.eye(4, dtype=jnp.float32), 32, axis=0) * isq)
    bmat = jnp.repeat(jnp.eye(4, dtype=jnp.float32), 32, axis=1)
    w1 = jnp.zeros((128, 128), jnp.float32)
    w1 = w1.at[0:_ED, 0:64].set(p['enc_w1']).at[_ED:2 * _ED, 64:128].set(
        p['enc_w1'])

    def body(ea_ref, kg_ref, vg_ref, qg_ref, w1r, b1r, w2r, b2r, wer, ber,
             g_ref, bb_ref, num_ref, den_ref):
        e = _leaky(jnp.dot(ea_ref[...], w1r[...],
                           preferred_element_type=jnp.float32) + b1r[...])
        e = jnp.dot(e, w2r[...], preferred_element_type=jnp.float32) + b2r[...]
        ek = jnp.dot(e, wer[...], preferred_element_type=jnp.float32) + ber[...]
        kj = kg_ref[...] + ek
        qk = qg_ref[...] * kj
        logits = jnp.dot(qk, g_ref[...], preferred_element_type=jnp.float32, precision=lax.Precision.HIGHEST)
        u = jnp.exp(logits)
        ub = jnp.dot(u, bb_ref[...], preferred_element_type=jnp.float32, precision=lax.Precision.HIGHEST)
        num_ref[...] = (vg_ref[...] + ek) * ub
        den_ref[...] = ub

    edge_spec = pl.BlockSpec((_BE2, 128), lambda i: (i, 0))
    edge_out = jax.ShapeDtypeStruct((_EP2, 128), jnp.float32)
    return pl.pallas_call(
        body,
        grid=(_EP2 // _BE2,),
        in_specs=[
            edge_spec, edge_spec, edge_spec, edge_spec,
            _full_spec((128, 128)), _full_spec((1, 128)),
            _full_spec((128, 128)), _full_spec((1, 128)),
            _full_spec((128, 128)), _full_spec((1, 128)),
            _full_spec((128, 4)), _full_spec((4, 128)),
        ],
        out_specs=[edge_spec, edge_spec],
        out_shape=[edge_out, edge_out],
    )(ea, kg, vg, qg,
      w1, _dup(p['enc_b1']), _bd(p['enc_w2']), _dup(p['enc_b2']),
      _bd(p['we']), _dup(p['be']), gmat, bmat)


def _tc_tr_node(h, num, den, p):
    wb = p['wbeta']
    bb2 = jnp.repeat(jnp.eye(2, dtype=jnp.float32), 64, axis=1)

    def bd2(w):
        m = jnp.zeros((128, 2), jnp.float32)
        return m.at[0:64, 0:1].set(w).at[64:128, 1:2].set(w)

    def body(h_ref, n_ref, d_ref, wskip, bskip, wb0, wb1, wb2, bb_ref, o_ref):
        den_s = d_ref[0] + d_ref[1]
        out = (n_ref[0] + n_ref[1]) / (den_s + 1e-16)
        x_r = jnp.dot(h_ref[...], wskip[...],
                      preferred_element_type=jnp.float32) + bskip[...]
        bl = (jnp.dot(out, wb0[...], preferred_element_type=jnp.float32, precision=lax.Precision.HIGHEST)
              + jnp.dot(x_r, wb1[...], preferred_element_type=jnp.float32, precision=lax.Precision.HIGHEST)
              + jnp.dot(out - x_r, wb2[...],
                        preferred_element_type=jnp.float32, precision=lax.Precision.HIGHEST))
        beta = _sigmoid(jnp.dot(bl, bb_ref[...],
                                preferred_element_type=jnp.float32, precision=lax.Precision.HIGHEST))
        o_ref[...] = beta * x_r + (1.0 - beta) * out

    return pl.pallas_call(
        body,
        grid=(_N // _BN,),
        in_specs=[
            pl.BlockSpec((_BN2, 128), lambda i: (i, 0)),
            pl.BlockSpec((2, _BN2, 128), lambda i: (0, i, 0)),
            pl.BlockSpec((2, _BN2, 128), lambda i: (0, i, 0)),
            _full_spec((128, 128)), _full_spec((1, 128)),
            _full_spec((128, 2)), _full_spec((128, 2)), _full_spec((128, 2)),
            _full_spec((2, 128)),
        ],
        out_specs=pl.BlockSpec((_BN2, 128), lambda i: (i, 0)),
        out_shape=jax.ShapeDtypeStruct((_N // 2, 128), jnp.float32),
    )(h, num, den, _bd(p['wskip']), _dup(p['bskip']),
      bd2(wb[0:_D]), bd2(wb[_D:2 * _D]), bd2(wb[2 * _D:3 * _D]), bb2)


def _tc_cls(sums, cnts, w1, b1, w2, b2):
    def body(s_ref, c_ref, w1r, b1r, w2r, b2r, o_ref):
        total = s_ref[pl.ds(0, _G)] + s_ref[pl.ds(_G, _G)]
        cnt = c_ref[pl.ds(0, _G), 0:1] + c_ref[pl.ds(_G, _G), 0:1]
        pooled = total / jnp.maximum(cnt, 1.0)
        z = _leaky(jnp.dot(pooled, w1r[...],
                           preferred_element_type=jnp.float32) + b1r[...])
        o_ref[...] = jnp.dot(z, w2r[...],
                             preferred_element_type=jnp.float32, precision=lax.Precision.HIGHEST) + b2r[...]

    return pl.pallas_call(
        body,
        in_specs=[
            _full_spec((2 * _G, _D)), _full_spec((2 * _G, 16)),
            _full_spec((_D, _D // 2)), _full_spec((1, _D // 2)),
            _full_spec((_D // 2, 10)), _full_spec((1, 10)),
        ],
        out_specs=_full_spec((_G, 10)),
        out_shape=jax.ShapeDtypeStruct((_G, 10), jnp.float32),
    )(sums, cnts, w1, b1.reshape(1, -1), w2, b2.reshape(1, -1))


def _gin_layer(p, h, ea, src2, dst_eo, zeros16):
    xg = _sc_gather64(h.reshape(_N, _D), src2)
    msg = _tc_gin_edge(ea, xg.reshape(_EP2, 128), _gin_edge_weights(p))
    aggr = _sc_scatter(msg, dst_eo, zeros16)
    return _tc_gin_node(h, aggr.reshape(2, _OP2, 128), p)


def kernel(x, edge_index, edge_attr, batch, params):
    src = edge_index[0]
    dst = edge_index[1]
    pad_e = _EP - _E
    zpad = jnp.zeros((pad_e,), jnp.int32)
    src2 = jnp.concatenate([src, zpad]).reshape(_ECH, 128)
    dst2g = jnp.concatenate([dst, zpad]).reshape(_ECH, 128)
    dst_eo = (jnp.concatenate([dst, jnp.full((pad_e,), _N, jnp.int32)])
              .reshape(_ECH, 64, 2).transpose(0, 2, 1).reshape(_ECH, 128))
    ea = jnp.pad(edge_attr, ((0, pad_e), (0, 0))).reshape(_EP2, 2 * _ED)
    ea = jnp.pad(ea, ((0, 0), (0, 128 - 2 * _ED)))
    zeros16 = jnp.zeros((_ACC_R // 8, 128), jnp.float32).reshape(_ACC_R, 16)
    zeros64 = jnp.zeros((_ACC_R // 16, 64), jnp.float32)
    ones16 = jnp.ones((128, 16), jnp.float32)

    in_w = params['in_w']
    w2c = jnp.zeros((2, 128), jnp.float32)
    w2c = w2c.at[0, 0:64].set(in_w[0]).at[1, 64:128].set(in_w[0])
    h = _tc_in(x.reshape(_N // 2, 2), w2c, _dup(params['in_b']))

    h = _gin_layer(params['gin1_0'], h, ea, src2, dst_eo, zeros16)
    h = _gin_layer(params['gin1_1'], h, ea, src2, dst_eo, zeros16)
    h = _tc_bn_leaky(h, params['bn1_g'], params['bn1_b'])
    res = h

    tr = params['tr']
    q, k, v = _tc_qkv(h, tr)
    qg = _sc_gather64(q.reshape(_N, _D), dst2g)
    kg = _sc_gather64(k.reshape(_N, _D), src2)
    vg = _sc_gather64(v.reshape(_N, _D), src2)
    num_e, den_e = _tc_tr_edge(ea, kg.reshape(_EP2, 128),
                               vg.reshape(_EP2, 128),
                               qg.reshape(_EP2, 128), tr)
    num = _sc_scatter(num_e, dst_eo, zeros16)
    den = _sc_scatter(den_e, dst_eo, zeros16)
    h = _tc_tr_node(h, num.reshape(2, _OP2, 128),
                    den.reshape(2, _OP2, 128), tr)
    h = _tc_bn_leaky(h, params['bntr_g'], params['bntr_b'], res)

    h = _gin_layer(params['gin2_0'], h, ea, src2, dst_eo, zeros16)
    h = _gin_layer(params['gin2_1'], h, ea, src2, dst_eo, zeros16)
    h = _tc_bn_leaky(h, params['bn2_g'], params['bn2_b'])

    hp = jnp.pad(h.reshape(_N, _D), ((0, _NP - _N), (0, 0)))
    bp = jnp.concatenate(
        [batch, jnp.full((_NP - _N,), _G, jnp.int32)]).reshape(_NCH, 128)
    sums, cnts = _sc_pool(hp, bp, zeros64, ones16)
    return _tc_cls(sums, cnts,
                   params['cls_w1'], params['cls_b1'],
                   params['cls_w2'], params['cls_b2'])
